# baseline (speedup 1.0000x reference)
import functools

import jax
import jax.numpy as jnp
from jax.experimental import pallas as pl
from jax.experimental.pallas import tpu as pltpu

N = 50000
R = 32
EMB = 16
HID = 64
E = 800000
B = 16384

ECOLS = 1024
EROWS = 784
EPAD = EROWS * ECOLS
RPB = 8
EGRID = EROWS // RPB

BCOLS = 1024
BROWS = B // BCOLS
BGRID = BROWS // RPB
BSTEP = RPB * BCOLS


def _row(ref, i):
    return ref[pl.ds(i, 1), :]


def _edge_rows(i, fn):
    for r in range(RPB):
        base = (i * RPB + r) * ECOLS
        n = jnp.clip(E - base, 0, ECOLS)

        def body(e, _, r=r):
            fn(r, e)
            return 0

        jax.lax.fori_loop(0, n, body, 0)


def _count_kernel(dst_ref, typ_ref, cnt_ref):
    i = pl.program_id(0)

    @pl.when(i == 0)
    def _zero():
        cnt_ref[...] = jnp.zeros_like(cnt_ref)

    lane = jax.lax.broadcasted_iota(jnp.int32, (1, R), 1)

    def edge(r, e):
        d = dst_ref[r, e]
        t = typ_ref[r, e]
        row = _row(cnt_ref, d)
        cnt_ref[pl.ds(d, 1), :] = row + (lane == t).astype(jnp.float32)

    _edge_rows(i, edge)


def _weights_kernel(dst_ref, typ_ref, cnt_ref, w_ref):
    i = pl.program_id(0)
    lane = jax.lax.broadcasted_iota(jnp.int32, (1, R), 1)

    def edge(r, e):
        d = dst_ref[r, e]
        t = typ_ref[r, e]
        row = _row(cnt_ref, d)
        c = jnp.sum(jnp.where(lane == t, row, 0.0))
        w_ref[r, e] = 1.0 / jnp.maximum(c, 1.0)

    _edge_rows(i, edge)


def _smem_blk():
    return pl.BlockSpec((RPB, ECOLS), lambda i: (i, 0),
                        memory_space=pltpu.SMEM)


def _full(shape):
    return pl.BlockSpec(shape, lambda i: (0,) * len(shape))


def _edge_weight_call(dst, typ):
    cnt = pl.pallas_call(
        _count_kernel,
        grid=(EGRID,),
        in_specs=[_smem_blk(), _smem_blk()],
        out_specs=_full((N, R)),
        out_shape=jax.ShapeDtypeStruct((N, R), jnp.float32),
    )(dst, typ)
    return pl.pallas_call(
        _weights_kernel,
        grid=(EGRID,),
        in_specs=[_smem_blk(), _smem_blk(), _full((N, R))],
        out_specs=_smem_blk(),
        out_shape=jax.ShapeDtypeStruct((EROWS, ECOLS), jnp.float32),
    )(dst, typ, cnt)


def _layer_kernel(src_ref, dst_ref, typ_ref, w_ref, x_ref, W_ref, root_ref,
                  b_ref, out_ref, *, din, apply_relu):
    i = pl.program_id(0)

    @pl.when(i == 0)
    def _init():
        out_ref[...] = (
            jnp.dot(x_ref[...], root_ref[...],
                    preferred_element_type=jnp.float32) + b_ref[...])

    def edge(r, e):
        s = src_ref[r, e]
        d = dst_ref[r, e]
        t = typ_ref[r, e]
        w = w_ref[r, e]
        xrow = _row(x_ref, s)
        Wt = W_ref[pl.ds(t, 1), :, :]
        msg = jnp.dot(xrow, Wt.reshape(din, HID),
                      preferred_element_type=jnp.float32)
        orow = _row(out_ref, d)
        out_ref[pl.ds(d, 1), :] = orow + w * msg

    _edge_rows(i, edge)

    if apply_relu:
        @pl.when(i == EGRID - 1)
        def _relu():
            out_ref[...] = jnp.maximum(out_ref[...], 0.0)


def _layer_call(src, dst, typ, w, x, W, root, b, apply_relu):
    din = x.shape[1]
    return pl.pallas_call(
        functools.partial(_layer_kernel, din=din, apply_relu=apply_relu),
        grid=(EGRID,),
        in_specs=[_smem_blk(), _smem_blk(), _smem_blk(), _smem_blk(),
                  _full((N, din)), _full((R, din, HID)),
                  _full((din, HID)), _full((1, HID))],
        out_specs=_full((N, HID)),
        out_shape=jax.ShapeDtypeStruct((N, HID), jnp.float32),
    )(src, dst, typ, w, x, W, root, b.reshape(1, HID))


def _classifier_kernel(d1_ref, d2_ref, x_ref, Wc1_ref, bc1_ref, Wc2_ref,
                       bc2_ref, out_ref, h_ref):
    for r in range(RPB):
        def body(e, _, r=r):
            i1 = d1_ref[r, e]
            i2 = d2_ref[r, e]
            o = r * BCOLS + e
            h_ref[pl.ds(o, 1), 0:HID] = _row(x_ref, i1)
            h_ref[pl.ds(o, 1), HID:2 * HID] = _row(x_ref, i2)
            return 0

        jax.lax.fori_loop(0, BCOLS, body, 0)

    h = jnp.maximum(
        jnp.dot(h_ref[...], Wc1_ref[...],
                preferred_element_type=jnp.float32) + bc1_ref[...], 0.0)
    out_ref[...] = (
        jnp.dot(h, Wc2_ref[...], preferred_element_type=jnp.float32)
        + bc2_ref[...])


def _classifier_call(d1, d2, x2, Wc1, bc1, Wc2, bc2):
    bblk = lambda: pl.BlockSpec((RPB, BCOLS), lambda i: (i, 0),
                                memory_space=pltpu.SMEM)
    return pl.pallas_call(
        _classifier_kernel,
        grid=(BGRID,),
        in_specs=[bblk(), bblk(), _full((N, HID)),
                  _full((2 * HID, HID)), _full((1, HID)),
                  _full((HID, R)), _full((1, R))],
        out_specs=pl.BlockSpec((BSTEP, R), lambda i: (i, 0)),
        out_shape=jax.ShapeDtypeStruct((B, R), jnp.float32),
        scratch_shapes=[pltpu.VMEM((BSTEP, 2 * HID), jnp.float32)],
    )(d1, d2, x2, Wc1, bc1.reshape(1, HID), Wc2, bc2.reshape(1, R))


def _pad_edges(a):
    return jnp.pad(a.astype(jnp.int32), (0, EPAD - E)).reshape(EROWS, ECOLS)


@jax.jit
def kernel(edge_index, edge_type, drug1_idx, drug2_idx, node_emb,
           W1, root1, b1, W2, root2, b2, Wc1, bc1, Wc2, bc2):
    src = _pad_edges(edge_index[0])
    dst = _pad_edges(edge_index[1])
    typ = _pad_edges(edge_type)
    w = _edge_weight_call(dst, typ)
    x1 = _layer_call(src, dst, typ, w, node_emb, W1, root1, b1,
                     apply_relu=True)
    x2 = _layer_call(src, dst, typ, w, x1, W2, root2, b2, apply_relu=False)
    return _classifier_call(drug1_idx.astype(jnp.int32).reshape(BROWS, BCOLS),
                            drug2_idx.astype(jnp.int32).reshape(BROWS, BCOLS),
                            x2, Wc1, bc1, Wc2, bc2)
